# Initial kernel scaffold; baseline (speedup 1.0000x reference)
import jax
import jax.numpy as jnp
from jax.experimental import pallas as pl


def kernel(x):
    raise NotImplementedError("write your pallas kernel here")


# speedup vs baseline: 6.8073x; 6.8073x over previous
import functools

import jax
import jax.numpy as jnp
from jax import lax
from jax.experimental import pallas as pl
from jax.experimental.pallas import tpu as pltpu
from jax.experimental.pallas import tpu_sc as plsc

B = 4096
L = 200
E = 64
REP = 4
NC = 2
NS = 16
NW = NC * NS
ROWS_PER_W = B // NW
CHUNKS = ROWS_PER_W // REP


def _img_body(o_ref):
    posi = lax.broadcasted_iota(jnp.int32, (L, E), 0)
    coli = lax.broadcasted_iota(jnp.int32, (L, E), 1)
    pos = posi.astype(jnp.float32)
    col = coli.astype(jnp.float32)
    angle = pos * jnp.exp(col * (-2.0 / E * jnp.log(10000.0)))
    t = jnp.where(coli % 2 == 0, jnp.sin(angle), jnp.cos(angle))
    t = jnp.where(posi == 0, 0.0, t) * (float(E) ** 0.5)
    o_ref[...] = jnp.broadcast_to(t[None], (REP, L, E))


def _table_image():
    return pl.pallas_call(
        _img_body,
        out_shape=jax.ShapeDtypeStruct((REP, L, E), jnp.float32),
    )()


_sc_mesh = plsc.VectorSubcoreMesh(core_axis_name="c", subcore_axis_name="s")


@functools.partial(
    pl.kernel,
    mesh=_sc_mesh,
    out_type=jax.ShapeDtypeStruct((B, L, E), jnp.float32),
    scratch_types=[
        pltpu.VMEM((REP, L, E), jnp.float32),
        pltpu.SemaphoreType.DMA,
    ],
)
def _broadcast_sc(img_hbm, out_hbm, img_v, sem):
    wid = lax.axis_index("s") * NC + lax.axis_index("c")
    pltpu.sync_copy(img_hbm, img_v)
    base = wid * ROWS_PER_W
    descs = [
        pltpu.async_copy(img_v, out_hbm.at[pl.ds(base + j * REP, REP)], sem)
        for j in range(CHUNKS)
    ]
    for d in descs:
        d.wait()


def kernel(x):
    assert x.shape == (B, L)
    return _broadcast_sc(_table_image())


# speedup vs baseline: 25.7285x; 3.7795x over previous
import functools

import jax
import jax.numpy as jnp
from jax import lax
from jax.experimental import pallas as pl
from jax.experimental.pallas import tpu as pltpu
from jax.experimental.pallas import tpu_sc as plsc

B = 4096
L = 200
E = 64
P = L * E
NC = 2
NS = 16
NW = NC * NS
PAIRS_PER_W = P // NW
WIMG = 256
NDMA = B // WIMG


def _seed_body(o_ref):
    f = lax.broadcasted_iota(jnp.int32, (P, 1), 0)
    posi = f // E
    coli = f % E
    pos = posi.astype(jnp.float32)
    col = coli.astype(jnp.float32)
    angle = pos * jnp.exp(col * (-2.0 / E * jnp.log(10000.0)))
    t = jnp.where(coli % 2 == 0, jnp.sin(angle), jnp.cos(angle))
    t = jnp.where(posi == 0, 0.0, t) * (float(E) ** 0.5)
    o_ref[...] = jnp.broadcast_to(t, (P, WIMG))


def _seed():
    return pl.pallas_call(
        _seed_body,
        out_shape=jax.ShapeDtypeStruct((P, WIMG), jnp.float32),
    )()


_sc_mesh = plsc.VectorSubcoreMesh(core_axis_name="c", subcore_axis_name="s")


@functools.partial(
    pl.kernel,
    mesh=_sc_mesh,
    out_type=jax.ShapeDtypeStruct((P, B), jnp.float32),
    scratch_types=[
        pltpu.VMEM((PAIRS_PER_W, WIMG), jnp.float32),
        pltpu.SemaphoreType.DMA,
    ],
)
def _broadcast_sc(seed_hbm, out_hbm, img_v, sem):
    wid = lax.axis_index("s") * NC + lax.axis_index("c")
    base = wid * PAIRS_PER_W
    pltpu.sync_copy(seed_hbm.at[pl.ds(base, PAIRS_PER_W)], img_v)
    descs = [
        pltpu.async_copy(
            img_v, out_hbm.at[pl.ds(base, PAIRS_PER_W), pl.ds(j * WIMG, WIMG)], sem
        )
        for j in range(NDMA)
    ]
    for d in descs:
        d.wait()


def kernel(x):
    assert x.shape == (B, L)
    out2 = _broadcast_sc(_seed())
    return out2.reshape(L, E, B).transpose(2, 0, 1)


# speedup vs baseline: 33.6170x; 1.3066x over previous
import functools

import jax
import jax.numpy as jnp
from jax import lax
from jax.experimental import pallas as pl
from jax.experimental.pallas import tpu as pltpu
from jax.experimental.pallas import tpu_sc as plsc

B = 4096
L = 200
E = 64
P = L * E
NC = 2
NS = 16
NW = NC * NS
PAIRS_PER_W = P // NW
WIMG = 256
NDMA = B // WIMG
LANES = 16


def _tbl_body(o_ref):
    r = lax.broadcasted_iota(jnp.int32, (NW, PAIRS_PER_W), 0)
    c = lax.broadcasted_iota(jnp.int32, (NW, PAIRS_PER_W), 1)
    f = r * PAIRS_PER_W + c
    posi = f // E
    coli = f % E
    pos = posi.astype(jnp.float32)
    col = coli.astype(jnp.float32)
    angle = pos * jnp.exp(col * (-2.0 / E * jnp.log(10000.0)))
    t = jnp.where(coli % 2 == 0, jnp.sin(angle), jnp.cos(angle))
    o_ref[...] = jnp.where(posi == 0, 0.0, t) * (float(E) ** 0.5)


def _table():
    return pl.pallas_call(
        _tbl_body,
        out_shape=jax.ShapeDtypeStruct((NW, PAIRS_PER_W), jnp.float32),
    )()


_sc_mesh = plsc.VectorSubcoreMesh(core_axis_name="c", subcore_axis_name="s")


@functools.partial(
    pl.kernel,
    mesh=_sc_mesh,
    out_type=jax.ShapeDtypeStruct((P, B), jnp.float32),
    scratch_types=[
        pltpu.VMEM((PAIRS_PER_W,), jnp.float32),
        pltpu.VMEM((PAIRS_PER_W, WIMG), jnp.float32),
        pltpu.SemaphoreType.DMA,
    ],
    compiler_params=pltpu.CompilerParams(needs_layout_passes=False),
)
def _broadcast_sc(tbl_hbm, out_hbm, tbl_v, img_v, sem):
    wid = lax.axis_index("s") * NC + lax.axis_index("c")
    base = wid * PAIRS_PER_W
    pltpu.sync_copy(tbl_hbm.at[wid], tbl_v)

    zeros16 = lax.broadcasted_iota(jnp.int32, (LANES,), 0) * 0

    def build_row(p, carry):
        vec = plsc.load_gather(tbl_v, [zeros16 + p])
        for j in range(WIMG // LANES):
            img_v[p, pl.ds(j * LANES, LANES)] = vec
        return carry

    lax.fori_loop(0, PAIRS_PER_W, build_row, 0)

    descs = [
        pltpu.async_copy(
            img_v, out_hbm.at[pl.ds(base, PAIRS_PER_W), pl.ds(j * WIMG, WIMG)], sem
        )
        for j in range(NDMA)
    ]
    for d in descs:
        d.wait()


def kernel(x):
    assert x.shape == (B, L)
    out2 = _broadcast_sc(_table())
    return out2.reshape(L, E, B).transpose(2, 0, 1)
